# baseline (speedup 1.0000x reference)
import jax
import jax.numpy as jnp
from jax.experimental import pallas as pl
from jax.experimental.pallas import tpu as pltpu

_K = 1024
_D = 64
_Q = 2
_TB = 1024


def _dot(a, b, dims):
    return jax.lax.dot_general(a, b, (dims, ((), ())),
                               preferred_element_type=jnp.float32,
                               precision=jax.lax.Precision.DEFAULT)


def _rvq_kernel(x_ref, cb_ref, sp_ref, cn_ref,
                out_ref, idx0_ref, idx1_ref, loss_ref):
    x = x_ref[...]
    residual = x
    quant_out = jnp.zeros_like(x)
    losses = []
    idx_refs = (idx0_ref, idx1_ref)
    for q in range(_Q):
        cb = cb_ref[0, q]
        cnorm = cn_ref[0, q]
        rnorm = jnp.sum(residual * residual, axis=1, keepdims=True)
        prod = _dot(residual, cb, ((1,), (1,)))
        d = (rnorm - 2.0 * prod) + cnorm[None, :]
        dmin = jnp.min(d, axis=1, keepdims=True)
        iota = jax.lax.broadcasted_iota(jnp.int32, d.shape, 1)
        idxc = jnp.min(jnp.where(d == dmin, iota, _K), axis=1, keepdims=True)
        idx_refs[q][0, 0] = idxc[:, 0]
        onehot = (iota == idxc).astype(jnp.float32)
        if q == 0:
            quant = ((_dot(onehot, sp_ref[0, 0], ((1,), (0,)))
                      + _dot(onehot, sp_ref[0, 1], ((1,), (0,))))
                     + _dot(onehot, sp_ref[0, 2], ((1,), (0,))))
        else:
            quant = _dot(onehot, cb, ((1,), (0,)))
        diff = quant - residual
        losses.append(jnp.sum(diff * diff))
        quant_out = quant_out + (residual + (quant - residual))
        residual = residual - quant
    out_ref[...] = quant_out
    loss_ref[0] = jnp.stack([jnp.broadcast_to(l, (128,)) for l in losses])


def _split3(cb):
    mask = jnp.int32(-65536)
    hi = jnp.bitwise_and(cb.view(jnp.int32), mask).view(jnp.float32)
    r = cb - hi
    mid = jnp.bitwise_and(r.view(jnp.int32), mask).view(jnp.float32)
    lo = r - mid
    return jnp.stack([hi, mid, lo])


def _rvq_combined(xa, xi, audio_codebooks, image_codebooks, interpret=False):
    na, d_ = xa.shape
    ni = xi.shape[0]
    ga, gi = na // _TB, ni // _TB
    g = ga + gi
    x = jnp.concatenate([xa, xi], axis=0)
    cbs = jnp.stack([audio_codebooks, image_codebooks])
    cns = jnp.sum(cbs * cbs, axis=-1)
    sps = jnp.stack([_split3(audio_codebooks[0]),
                     _split3(image_codebooks[0])])

    def _sel(i):
        return jnp.where(i < ga, 0, 1)

    out, idx0, idx1, lossp = pl.pallas_call(
        _rvq_kernel,
        grid=(g,),
        in_specs=[
            pl.BlockSpec((_TB, d_), lambda i: (i, 0)),
            pl.BlockSpec((1, _Q, _K, d_), lambda i: (_sel(i), 0, 0, 0)),
            pl.BlockSpec((1, 3, _K, d_), lambda i: (_sel(i), 0, 0, 0)),
            pl.BlockSpec((1, _Q, _K), lambda i: (_sel(i), 0, 0)),
        ],
        out_specs=[
            pl.BlockSpec((_TB, d_), lambda i: (i, 0)),
            pl.BlockSpec((1, 1, _TB), lambda i: (i, 0, 0)),
            pl.BlockSpec((1, 1, _TB), lambda i: (i, 0, 0)),
            pl.BlockSpec((1, _Q, 128), lambda i: (i, 0, 0)),
        ],
        out_shape=[
            jax.ShapeDtypeStruct((na + ni, d_), jnp.float32),
            jax.ShapeDtypeStruct((g, 1, _TB), jnp.int32),
            jax.ShapeDtypeStruct((g, 1, _TB), jnp.int32),
            jax.ShapeDtypeStruct((g, _Q, 128), jnp.float32),
        ],
        compiler_params=pltpu.CompilerParams(
            dimension_semantics=("arbitrary",)),
        interpret=interpret,
    )(x, cbs, sps, cns)
    qa, qi_ = out[:na], out[na:]
    ia = jnp.stack([idx0[:ga].reshape(na), idx1[:ga].reshape(na)], axis=-1)
    ii = jnp.stack([idx0[ga:].reshape(ni), idx1[ga:].reshape(ni)], axis=-1)
    la = lossp[:ga, :, 0].sum(axis=0) / (na * d_)
    li = lossp[ga:, :, 0].sum(axis=0) / (ni * d_)
    return qa, qi_, ia, ii, la, li


def kernel(audio_input, image_input, audio_codebooks, image_codebooks):
    ba, da, ta = audio_input.shape
    xa = jnp.transpose(audio_input, (0, 2, 1)).reshape(ba * ta, da)
    bi, di, h, w = image_input.shape
    xi = jnp.transpose(image_input, (0, 2, 3, 1)).reshape(bi * h * w, di)

    qa, qi, ia, ii, la, li = _rvq_combined(xa, xi, audio_codebooks, image_codebooks)

    recon_audio = jnp.transpose(qa.reshape(ba, ta, da), (0, 2, 1))
    recon_image = jnp.transpose(qi.reshape(bi, h, w, di), (0, 3, 1, 2))
    audio_indices = ia.reshape(ba, ta, _Q)
    image_indices = ii.reshape(bi, h * w, _Q)
    return (recon_audio, recon_image, la, li, audio_indices, image_indices)


# speedup vs baseline: 1.1163x; 1.1163x over previous
import jax
import jax.numpy as jnp
from jax.experimental import pallas as pl
from jax.experimental.pallas import tpu as pltpu

_K = 1024
_D = 64
_Q = 2


def _dot(a, b, dims):
    return jax.lax.dot_general(a, b, (dims, ((), ())),
                               preferred_element_type=jnp.float32,
                               precision=jax.lax.Precision.DEFAULT)


def _rvq_kernel(x_ref, cb_ref, sp_ref, cn_ref,
                out_ref, idx0_ref, idx1_ref, loss_ref):
    x = jnp.transpose(x_ref[0], (1, 0))
    residual = x
    quant_out = jnp.zeros_like(x)
    losses = []
    idx_refs = (idx0_ref, idx1_ref)
    for q in range(_Q):
        cb = cb_ref[q]
        cnorm = cn_ref[q]
        rnorm = jnp.sum(residual * residual, axis=1, keepdims=True)
        prod = _dot(residual, cb, ((1,), (1,)))
        d = (rnorm - 2.0 * prod) + cnorm[None, :]
        dmin = jnp.min(d, axis=1, keepdims=True)
        iota = jax.lax.broadcasted_iota(jnp.int32, d.shape, 1)
        idxc = jnp.min(jnp.where(d == dmin, iota, _K), axis=1, keepdims=True)
        idx_refs[q][0, 0] = idxc[:, 0]
        onehot = (iota == idxc).astype(jnp.float32)
        if q == 0:
            quant = ((_dot(onehot, sp_ref[0], ((1,), (0,)))
                      + _dot(onehot, sp_ref[1], ((1,), (0,))))
                     + _dot(onehot, sp_ref[2], ((1,), (0,))))
        else:
            quant = _dot(onehot, cb, ((1,), (0,)))
        diff = quant - residual
        losses.append(jnp.sum(diff * diff))
        quant_out = quant_out + (residual + (quant - residual))
        residual = residual - quant
    out_ref[0] = jnp.transpose(quant_out, (1, 0))
    loss_ref[0] = jnp.stack([jnp.broadcast_to(l, (128,)) for l in losses])


def _split3(cb):
    mask = jnp.int32(-65536)
    hi = jnp.bitwise_and(cb.view(jnp.int32), mask).view(jnp.float32)
    r = cb - hi
    mid = jnp.bitwise_and(r.view(jnp.int32), mask).view(jnp.float32)
    lo = r - mid
    return jnp.stack([hi, mid, lo])


def _rvq(x, codebooks, tb, interpret=False):
    b, d_, t = x.shape
    g = t // tb
    cnorm = jnp.sum(codebooks * codebooks, axis=-1)
    out, idx0, idx1, lossp = pl.pallas_call(
        _rvq_kernel,
        grid=(b, g),
        in_specs=[
            pl.BlockSpec((1, d_, tb), lambda i, j: (i, 0, j)),
            pl.BlockSpec((_Q, _K, d_), lambda i, j: (0, 0, 0)),
            pl.BlockSpec((3, _K, d_), lambda i, j: (0, 0, 0)),
            pl.BlockSpec((_Q, _K), lambda i, j: (0, 0)),
        ],
        out_specs=[
            pl.BlockSpec((1, d_, tb), lambda i, j: (i, 0, j)),
            pl.BlockSpec((1, 1, tb), lambda i, j: (i * g + j, 0, 0)),
            pl.BlockSpec((1, 1, tb), lambda i, j: (i * g + j, 0, 0)),
            pl.BlockSpec((1, _Q, 128), lambda i, j: (i * g + j, 0, 0)),
        ],
        out_shape=[
            jax.ShapeDtypeStruct((b, d_, t), jnp.float32),
            jax.ShapeDtypeStruct((b * g, 1, tb), jnp.int32),
            jax.ShapeDtypeStruct((b * g, 1, tb), jnp.int32),
            jax.ShapeDtypeStruct((b * g, _Q, 128), jnp.float32),
        ],
        compiler_params=pltpu.CompilerParams(
            dimension_semantics=("parallel", "parallel")),
        interpret=interpret,
    )(x, codebooks, _split3(codebooks[0]), cnorm)
    indices = jnp.stack([idx0.reshape(b, t), idx1.reshape(b, t)], axis=-1)
    loss = lossp[:, :, 0].sum(axis=0) / (b * t * d_)
    return out, indices, loss


def kernel(audio_input, image_input, audio_codebooks, image_codebooks):
    recon_audio, audio_indices, vq_audio_loss = _rvq(
        audio_input, audio_codebooks, 1024)

    bi, di, h, w = image_input.shape
    xi = image_input.reshape(bi, di, h * w)
    recon_img, image_indices, vq_image_loss = _rvq(xi, image_codebooks, 1024)
    recon_image = recon_img.reshape(bi, di, h, w)

    return (recon_audio, recon_image, vq_audio_loss, vq_image_loss,
            audio_indices, image_indices)
